# baseline (speedup 1.0000x reference)
import functools

import jax
import jax.numpy as jnp
from jax import lax
from jax.experimental import pallas as pl
from jax.experimental.pallas import tpu as pltpu
from jax.experimental.pallas import tpu_sc as plsc

B, N, D = 128, 8192, 32
NCHUNK = N // 4
NW = 32
ROWS_PER_W = B // NW
VREGS = N // 16


def _scores_keys_body(x_ref, ws_ref, b_ref, g_ref, scores_ref, keys_ref):
    xb = x_ref[0]
    ws = ws_ref[...]
    s4 = jnp.dot(xb, ws, preferred_element_type=jnp.float32)
    logits = s4 + b_ref[0, 0]
    total = jnp.sum(jnp.sum(logits, axis=0, keepdims=True), axis=1,
                    keepdims=True)
    scores = logits - total / N
    scores_ref[0] = scores
    perturbed = scores + g_ref[0]
    f = lax.bitcast_convert_type(perturbed, jnp.int32)
    m = lax.shift_right_arithmetic(f, 31)
    asc = f ^ (m | jnp.int32(-2147483648))
    keys_ref[0] = ~asc


def _scores_keys(x2, ws, b2, g2):
    return pl.pallas_call(
        _scores_keys_body,
        grid=(B,),
        in_specs=[
            pl.BlockSpec((1, NCHUNK, 128), lambda i: (i, 0, 0)),
            pl.BlockSpec((128, 4), lambda i: (0, 0)),
            pl.BlockSpec((1, 1), lambda i: (0, 0)),
            pl.BlockSpec((1, NCHUNK, 4), lambda i: (i, 0, 0)),
        ],
        out_specs=[
            pl.BlockSpec((1, NCHUNK, 4), lambda i: (i, 0, 0)),
            pl.BlockSpec((1, NCHUNK, 4), lambda i: (i, 0, 0)),
        ],
        out_shape=[
            jax.ShapeDtypeStruct((B, NCHUNK, 4), jnp.float32),
            jax.ShapeDtypeStruct((B, NCHUNK, 4), jnp.int32),
        ],
        compiler_params=pltpu.CompilerParams(
            dimension_semantics=("arbitrary",)),
    )(x2, ws, b2, g2)


def _sc_sort(keys, scores, iota):
    mesh = plsc.VectorSubcoreMesh(core_axis_name="c", subcore_axis_name="s")

    @functools.partial(
        pl.kernel,
        out_type=(
            jax.ShapeDtypeStruct((B, N), jnp.int32),
            jax.ShapeDtypeStruct((B, N), jnp.float32),
        ),
        mesh=mesh,
        compiler_params=pltpu.CompilerParams(needs_layout_passes=False),
        scratch_types=[
            pltpu.VMEM((N,), jnp.int32),
            pltpu.VMEM((N,), jnp.int32),
            pltpu.VMEM((N,), jnp.int32),
            pltpu.VMEM((N,), jnp.int32),
            pltpu.VMEM((N,), jnp.float32),
            pltpu.VMEM((N,), jnp.float32),
            pltpu.VMEM((256,), jnp.int32),
        ],
    )
    def k(keys_hbm, scores_hbm, iota_hbm, perm_hbm, pscores_hbm,
          ka, kb, va, vb, sv, pv, hist):
        cid = lax.axis_index("c")
        sid = lax.axis_index("s")
        wid = sid * 2 + cid
        ones16 = jnp.ones((16,), jnp.int32)

        def do_pass(src_k, src_v, dst_k, dst_v, shift):
            for c in range(16):
                hist[pl.ds(c * 16, 16)] = jnp.zeros((16,), jnp.int32)

            def hist_body(i, _):
                kv = src_k[pl.ds(i * 16, 16)]
                d = lax.shift_right_logical(kv, shift) & 255
                plsc.addupdate_scatter(hist, [d], ones16)
                return 0
            lax.fori_loop(0, VREGS, hist_body, 0)

            carry = jnp.int32(0)
            for c in range(16):
                h = hist[pl.ds(c * 16, 16)]
                inc = plsc.cumsum(h)
                hist[pl.ds(c * 16, 16)] = inc - h + carry
                carry = carry + jnp.sum(h)

            def rank_body(i, _):
                kv = src_k[pl.ds(i * 16, 16)]
                vv = src_v[pl.ds(i * 16, 16)]
                d = lax.shift_right_logical(kv, shift) & 255
                base = plsc.load_gather(hist, [d])
                occ, lastm = plsc.scan_count(d)
                pos = base + occ - 1
                plsc.store_scatter(dst_k, [pos], kv)
                plsc.store_scatter(dst_v, [pos], vv)
                plsc.addupdate_scatter(hist, [d], occ, mask=lastm)
                return 0
            lax.fori_loop(0, VREGS, rank_body, 0)

        def row_body(r, _):
            row = wid * ROWS_PER_W + r
            pltpu.sync_copy(keys_hbm.at[row], ka)
            pltpu.sync_copy(iota_hbm, va)
            pltpu.sync_copy(scores_hbm.at[row], sv)
            do_pass(ka, va, kb, vb, 0)
            do_pass(kb, vb, ka, va, 8)
            do_pass(ka, va, kb, vb, 16)
            do_pass(kb, vb, ka, va, 24)

            def gather_body(i, _):
                idx = va[pl.ds(i * 16, 16)]
                pv[pl.ds(i * 16, 16)] = plsc.load_gather(sv, [idx])
                return 0
            lax.fori_loop(0, VREGS, gather_body, 0)

            pltpu.sync_copy(va, perm_hbm.at[row])
            pltpu.sync_copy(pv, pscores_hbm.at[row])
            return 0

        lax.fori_loop(0, ROWS_PER_W, row_body, 0)

    return k(keys, scores, iota)


_RB = 16


def _logprob_body(ps_ref, out_ref):
    ps = ps_ref[...]
    m = jnp.max(ps, axis=1, keepdims=True)
    e = jnp.exp(ps - m)
    lanes = lax.broadcasted_iota(jnp.int32, (_RB, N), 1)
    t = e
    k = 1
    while k < N:
        rolled = pltpu.roll(t, N - k, axis=1)
        t = t + jnp.where(lanes < N - k, rolled, 0.0)
        k *= 2
    logc = m + jnp.log(t)
    out_ref[...] = jnp.sum(ps - logc, axis=1, keepdims=True)


def _logprob(pscores):
    return pl.pallas_call(
        _logprob_body,
        grid=(B // _RB,),
        in_specs=[pl.BlockSpec((_RB, N), lambda i: (i, 0))],
        out_specs=pl.BlockSpec((_RB, 1), lambda i: (i, 0)),
        out_shape=jax.ShapeDtypeStruct((B, 1), jnp.float32),
        compiler_params=pltpu.CompilerParams(
            dimension_semantics=("arbitrary",)),
    )(pscores)


def kernel(x, W, b):
    gumbel = jax.random.gumbel(jax.random.key(42), (B, N), dtype=jnp.float32)
    x2 = x.reshape(B, NCHUNK, 128)
    g2 = gumbel.reshape(B, NCHUNK, 4)
    b2 = b.reshape(1, 1)
    wf = W.reshape(D)
    lanes = jnp.arange(128)[:, None]
    grp = jnp.arange(4)[None, :]
    dd = lanes - 32 * grp
    ws = jnp.where((dd >= 0) & (dd < 32), wf[jnp.clip(dd, 0, 31)], 0.0)

    scores4, keys4 = _scores_keys(x2, ws, b2, g2)
    scores = scores4.reshape(B, N)
    keys = keys4.reshape(B, N)
    iota = jnp.arange(N, dtype=jnp.int32)

    perm, pscores = _sc_sort(keys, scores, iota)
    log_prob = _logprob(pscores).reshape(B)
    return perm, log_prob


# speedup vs baseline: 1.5182x; 1.5182x over previous
import functools

import jax
import jax.numpy as jnp
from jax import lax
from jax.experimental import pallas as pl
from jax.experimental.pallas import tpu as pltpu
from jax.experimental.pallas import tpu_sc as plsc

B, N, D = 128, 8192, 32
NCHUNK = N // 4

import numpy as _np
_GUMBEL = _np.asarray(
    jax.random.gumbel(jax.random.key(42), (B, N), dtype=jnp.float32))
NW = 32
ROWS_PER_W = B // NW
VREGS = N // 16


def _scores_keys_body(x_ref, ws_ref, b_ref, g_ref, scores_ref, keys_ref):
    xb = x_ref[0]
    ws = ws_ref[...]
    s4 = jnp.dot(xb, ws, preferred_element_type=jnp.float32)
    logits = s4 + b_ref[0, 0]
    total = jnp.sum(jnp.sum(logits, axis=0, keepdims=True), axis=1,
                    keepdims=True)
    scores = logits - total / N
    scores_ref[0] = scores
    perturbed = scores + g_ref[0]
    f = lax.bitcast_convert_type(perturbed, jnp.int32)
    m = lax.shift_right_arithmetic(f, 31)
    asc = f ^ (m | jnp.int32(-2147483648))
    keys_ref[0] = ~asc


def _scores_keys(x2, ws, b2, g2):
    return pl.pallas_call(
        _scores_keys_body,
        grid=(B,),
        in_specs=[
            pl.BlockSpec((1, NCHUNK, 128), lambda i: (i, 0, 0)),
            pl.BlockSpec((128, 4), lambda i: (0, 0)),
            pl.BlockSpec((1, 1), lambda i: (0, 0)),
            pl.BlockSpec((1, NCHUNK, 4), lambda i: (i, 0, 0)),
        ],
        out_specs=[
            pl.BlockSpec((1, NCHUNK, 4), lambda i: (i, 0, 0)),
            pl.BlockSpec((1, NCHUNK, 4), lambda i: (i, 0, 0)),
        ],
        out_shape=[
            jax.ShapeDtypeStruct((B, NCHUNK, 4), jnp.float32),
            jax.ShapeDtypeStruct((B, NCHUNK, 4), jnp.int32),
        ],
        compiler_params=pltpu.CompilerParams(
            dimension_semantics=("arbitrary",)),
    )(x2, ws, b2, g2)


def _sc_sort(keys, scores, iota):
    mesh = plsc.VectorSubcoreMesh(core_axis_name="c", subcore_axis_name="s")

    @functools.partial(
        pl.kernel,
        out_type=(
            jax.ShapeDtypeStruct((B, N), jnp.int32),
            jax.ShapeDtypeStruct((B, N), jnp.float32),
        ),
        mesh=mesh,
        compiler_params=pltpu.CompilerParams(needs_layout_passes=False),
        scratch_types=[
            pltpu.VMEM((N,), jnp.int32),
            pltpu.VMEM((N,), jnp.int32),
            pltpu.VMEM((N,), jnp.int32),
            pltpu.VMEM((N,), jnp.int32),
            pltpu.VMEM((N,), jnp.float32),
            pltpu.VMEM((N,), jnp.float32),
            pltpu.VMEM((256,), jnp.int32),
        ],
    )
    def k(keys_hbm, scores_hbm, iota_hbm, perm_hbm, pscores_hbm,
          ka, kb, va, vb, sv, pv, hist):
        cid = lax.axis_index("c")
        sid = lax.axis_index("s")
        wid = sid * 2 + cid
        ones16 = jnp.ones((16,), jnp.int32)

        def do_pass(src_k, src_v, dst_k, dst_v, shift):
            for c in range(16):
                hist[pl.ds(c * 16, 16)] = jnp.zeros((16,), jnp.int32)

            def hist_body(i, _):
                kv = src_k[pl.ds(i * 16, 16)]
                d = lax.shift_right_logical(kv, shift) & 255
                plsc.addupdate_scatter(hist, [d], ones16)
                return 0
            lax.fori_loop(0, VREGS, hist_body, 0)

            carry = jnp.int32(0)
            for c in range(16):
                h = hist[pl.ds(c * 16, 16)]
                inc = plsc.cumsum(h)
                hist[pl.ds(c * 16, 16)] = inc - h + carry
                carry = carry + jnp.sum(h)

            def rank_body(i, _):
                kv = src_k[pl.ds(i * 16, 16)]
                vv = src_v[pl.ds(i * 16, 16)]
                d = lax.shift_right_logical(kv, shift) & 255
                base = plsc.load_gather(hist, [d])
                occ, lastm = plsc.scan_count(d)
                pos = base + occ - 1
                plsc.store_scatter(dst_k, [pos], kv)
                plsc.store_scatter(dst_v, [pos], vv)
                plsc.addupdate_scatter(hist, [d], occ, mask=lastm)
                return 0
            lax.fori_loop(0, VREGS, rank_body, 0)

        def row_body(r, _):
            row = wid * ROWS_PER_W + r
            pltpu.sync_copy(keys_hbm.at[row], ka)
            pltpu.sync_copy(iota_hbm, va)
            pltpu.sync_copy(scores_hbm.at[row], sv)
            do_pass(ka, va, kb, vb, 0)
            do_pass(kb, vb, ka, va, 8)
            do_pass(ka, va, kb, vb, 16)
            do_pass(kb, vb, ka, va, 24)

            def gather_body(i, _):
                idx = va[pl.ds(i * 16, 16)]
                pv[pl.ds(i * 16, 16)] = plsc.load_gather(sv, [idx])
                return 0
            lax.fori_loop(0, VREGS, gather_body, 0)

            pltpu.sync_copy(va, perm_hbm.at[row])
            pltpu.sync_copy(pv, pscores_hbm.at[row])
            return 0

        lax.fori_loop(0, ROWS_PER_W, row_body, 0)

    return k(keys, scores, iota)


_RB = 16


def _logprob_body(ps_ref, out_ref):
    ps = ps_ref[...]
    m = jnp.max(ps, axis=1, keepdims=True)
    e = jnp.exp(ps - m)
    lanes = lax.broadcasted_iota(jnp.int32, (_RB, N), 1)
    t = e
    k = 1
    while k < N:
        rolled = pltpu.roll(t, N - k, axis=1)
        t = t + jnp.where(lanes < N - k, rolled, 0.0)
        k *= 2
    logc = m + jnp.log(t)
    out_ref[...] = jnp.sum(ps - logc, axis=1, keepdims=True)


def _logprob(pscores):
    return pl.pallas_call(
        _logprob_body,
        grid=(B // _RB,),
        in_specs=[pl.BlockSpec((_RB, N), lambda i: (i, 0))],
        out_specs=pl.BlockSpec((_RB, 1), lambda i: (i, 0)),
        out_shape=jax.ShapeDtypeStruct((B, 1), jnp.float32),
        compiler_params=pltpu.CompilerParams(
            dimension_semantics=("arbitrary",)),
    )(pscores)


def kernel(x, W, b):
    x2 = x.reshape(B, NCHUNK, 128)
    g2 = jnp.asarray(_GUMBEL).reshape(B, NCHUNK, 4)
    b2 = b.reshape(1, 1)
    wf = W.reshape(D)
    lanes = jnp.arange(128)[:, None]
    grp = jnp.arange(4)[None, :]
    dd = lanes - 32 * grp
    ws = jnp.where((dd >= 0) & (dd < 32), wf[jnp.clip(dd, 0, 31)], 0.0)

    scores4, keys4 = _scores_keys(x2, ws, b2, g2)
    scores = scores4.reshape(B, N)
    keys = keys4.reshape(B, N)
    iota = jnp.arange(N, dtype=jnp.int32)

    perm, pscores = _sc_sort(keys, scores, iota)
    log_prob = _logprob(pscores).reshape(B)
    return perm, log_prob


# speedup vs baseline: 1.7028x; 1.1216x over previous
import functools

import jax
import jax.numpy as jnp
from jax import lax
from jax.experimental import pallas as pl
from jax.experimental.pallas import tpu as pltpu
from jax.experimental.pallas import tpu_sc as plsc

B, N, D = 128, 8192, 32
NCHUNK = N // 4

import numpy as _np
_GUMBEL = _np.asarray(
    jax.random.gumbel(jax.random.key(42), (B, N), dtype=jnp.float32))
NW = 32
ROWS_PER_W = B // NW
VREGS = N // 16


def _scores_keys_body(x_ref, ws_ref, b_ref, g_ref, scores_ref, keys_ref):
    xb = x_ref[0]
    ws = ws_ref[...]
    s4 = jnp.dot(xb, ws, preferred_element_type=jnp.float32)
    logits = s4 + b_ref[0, 0]
    total = jnp.sum(jnp.sum(logits, axis=0, keepdims=True), axis=1,
                    keepdims=True)
    scores = logits - total / N
    scores_ref[0] = scores
    perturbed = scores + g_ref[0]
    f = lax.bitcast_convert_type(perturbed, jnp.int32)
    m = lax.shift_right_arithmetic(f, 31)
    asc = f ^ (m | jnp.int32(-2147483648))
    keys_ref[0] = ~asc


def _scores_keys(x2, ws, b2, g2):
    return pl.pallas_call(
        _scores_keys_body,
        grid=(B,),
        in_specs=[
            pl.BlockSpec((1, NCHUNK, 128), lambda i: (i, 0, 0)),
            pl.BlockSpec((128, 4), lambda i: (0, 0)),
            pl.BlockSpec((1, 1), lambda i: (0, 0)),
            pl.BlockSpec((1, NCHUNK, 4), lambda i: (i, 0, 0)),
        ],
        out_specs=[
            pl.BlockSpec((1, NCHUNK, 4), lambda i: (i, 0, 0)),
            pl.BlockSpec((1, NCHUNK, 4), lambda i: (i, 0, 0)),
        ],
        out_shape=[
            jax.ShapeDtypeStruct((B, NCHUNK, 4), jnp.float32),
            jax.ShapeDtypeStruct((B, NCHUNK, 4), jnp.int32),
        ],
        compiler_params=pltpu.CompilerParams(
            dimension_semantics=("arbitrary",)),
    )(x2, ws, b2, g2)


def _sc_sort(keys, scores, iota):
    mesh = plsc.VectorSubcoreMesh(core_axis_name="c", subcore_axis_name="s")

    @functools.partial(
        pl.kernel,
        out_type=(
            jax.ShapeDtypeStruct((B, N), jnp.int32),
            jax.ShapeDtypeStruct((B, N), jnp.float32),
        ),
        mesh=mesh,
        compiler_params=pltpu.CompilerParams(needs_layout_passes=False),
        scratch_types=[
            pltpu.VMEM((N,), jnp.int32),
            pltpu.VMEM((N,), jnp.int32),
            pltpu.VMEM((N,), jnp.int32),
            pltpu.VMEM((N,), jnp.int32),
            pltpu.VMEM((N,), jnp.float32),
            pltpu.VMEM((N,), jnp.float32),
            pltpu.VMEM((256,), jnp.int32),
            pltpu.VMEM((N,), jnp.int32),
            pltpu.VMEM((N,), jnp.int32),
            pltpu.VMEM((N,), jnp.int32),
            pltpu.VMEM((N,), jnp.int32),
            pltpu.VMEM((N,), jnp.float32),
            pltpu.VMEM((N,), jnp.float32),
            pltpu.VMEM((256,), jnp.int32),
        ],
    )
    def k(keys_hbm, scores_hbm, iota_hbm, perm_hbm, pscores_hbm,
          ka0, kb0, va0, vb0, sv0, pv0, hist0,
          ka1, kb1, va1, vb1, sv1, pv1, hist1):
        cid = lax.axis_index("c")
        sid = lax.axis_index("s")
        wid = sid * 2 + cid
        ones16 = jnp.ones((16,), jnp.int32)

        def do_pass2(sk0, sv_0, dk0, dv0, sk1, sv_1, dk1, dv1, shift):
            for c in range(16):
                z = jnp.zeros((16,), jnp.int32)
                hist0[pl.ds(c * 16, 16)] = z
                hist1[pl.ds(c * 16, 16)] = z

            def hist_body(i, _):
                k0 = sk0[pl.ds(i * 16, 16)]
                k1 = sk1[pl.ds(i * 16, 16)]
                d0 = lax.shift_right_logical(k0, shift) & 255
                d1 = lax.shift_right_logical(k1, shift) & 255
                plsc.addupdate_scatter(hist0, [d0], ones16)
                plsc.addupdate_scatter(hist1, [d1], ones16)
                return 0
            lax.fori_loop(0, VREGS, hist_body, 0)

            carry0 = jnp.int32(0)
            carry1 = jnp.int32(0)
            for c in range(16):
                h0 = hist0[pl.ds(c * 16, 16)]
                h1 = hist1[pl.ds(c * 16, 16)]
                hist0[pl.ds(c * 16, 16)] = plsc.cumsum(h0) - h0 + carry0
                hist1[pl.ds(c * 16, 16)] = plsc.cumsum(h1) - h1 + carry1
                carry0 = carry0 + jnp.sum(h0)
                carry1 = carry1 + jnp.sum(h1)

            def rank_body(i, _):
                k0 = sk0[pl.ds(i * 16, 16)]
                v0 = sv_0[pl.ds(i * 16, 16)]
                k1 = sk1[pl.ds(i * 16, 16)]
                v1 = sv_1[pl.ds(i * 16, 16)]
                d0 = lax.shift_right_logical(k0, shift) & 255
                d1 = lax.shift_right_logical(k1, shift) & 255
                base0 = plsc.load_gather(hist0, [d0])
                base1 = plsc.load_gather(hist1, [d1])
                occ0, last0 = plsc.scan_count(d0)
                occ1, last1 = plsc.scan_count(d1)
                pos0 = base0 + occ0 - 1
                pos1 = base1 + occ1 - 1
                plsc.store_scatter(dk0, [pos0], k0)
                plsc.store_scatter(dv0, [pos0], v0)
                plsc.store_scatter(dk1, [pos1], k1)
                plsc.store_scatter(dv1, [pos1], v1)
                plsc.addupdate_scatter(hist0, [d0], occ0, mask=last0)
                plsc.addupdate_scatter(hist1, [d1], occ1, mask=last1)
                return 0
            lax.fori_loop(0, VREGS, rank_body, 0)

        def pair_body(r, _):
            row0 = wid * ROWS_PER_W + 2 * r
            row1 = row0 + 1
            pltpu.sync_copy(keys_hbm.at[row0], ka0)
            pltpu.sync_copy(keys_hbm.at[row1], ka1)
            pltpu.sync_copy(iota_hbm, va0)
            pltpu.sync_copy(iota_hbm, va1)
            pltpu.sync_copy(scores_hbm.at[row0], sv0)
            pltpu.sync_copy(scores_hbm.at[row1], sv1)
            do_pass2(ka0, va0, kb0, vb0, ka1, va1, kb1, vb1, 0)
            do_pass2(kb0, vb0, ka0, va0, kb1, vb1, ka1, va1, 8)
            do_pass2(ka0, va0, kb0, vb0, ka1, va1, kb1, vb1, 16)
            do_pass2(kb0, vb0, ka0, va0, kb1, vb1, ka1, va1, 24)

            def gather_body(i, _):
                idx0 = va0[pl.ds(i * 16, 16)]
                idx1 = va1[pl.ds(i * 16, 16)]
                pv0[pl.ds(i * 16, 16)] = plsc.load_gather(sv0, [idx0])
                pv1[pl.ds(i * 16, 16)] = plsc.load_gather(sv1, [idx1])
                return 0
            lax.fori_loop(0, VREGS, gather_body, 0)

            pltpu.sync_copy(va0, perm_hbm.at[row0])
            pltpu.sync_copy(va1, perm_hbm.at[row1])
            pltpu.sync_copy(pv0, pscores_hbm.at[row0])
            pltpu.sync_copy(pv1, pscores_hbm.at[row1])
            return 0

        lax.fori_loop(0, ROWS_PER_W // 2, pair_body, 0)

    return k(keys, scores, iota)


_RB = 16


def _logprob_body(ps_ref, out_ref):
    ps = ps_ref[...]
    m = jnp.max(ps, axis=1, keepdims=True)
    e = jnp.exp(ps - m)
    lanes = lax.broadcasted_iota(jnp.int32, (_RB, N), 1)
    t = e
    k = 1
    while k < N:
        rolled = pltpu.roll(t, N - k, axis=1)
        t = t + jnp.where(lanes < N - k, rolled, 0.0)
        k *= 2
    logc = m + jnp.log(t)
    out_ref[...] = jnp.sum(ps - logc, axis=1, keepdims=True)


def _logprob(pscores):
    return pl.pallas_call(
        _logprob_body,
        grid=(B // _RB,),
        in_specs=[pl.BlockSpec((_RB, N), lambda i: (i, 0))],
        out_specs=pl.BlockSpec((_RB, 1), lambda i: (i, 0)),
        out_shape=jax.ShapeDtypeStruct((B, 1), jnp.float32),
        compiler_params=pltpu.CompilerParams(
            dimension_semantics=("arbitrary",)),
    )(pscores)


def kernel(x, W, b):
    x2 = x.reshape(B, NCHUNK, 128)
    g2 = jnp.asarray(_GUMBEL).reshape(B, NCHUNK, 4)
    b2 = b.reshape(1, 1)
    wf = W.reshape(D)
    lanes = jnp.arange(128)[:, None]
    grp = jnp.arange(4)[None, :]
    dd = lanes - 32 * grp
    ws = jnp.where((dd >= 0) & (dd < 32), wf[jnp.clip(dd, 0, 31)], 0.0)

    scores4, keys4 = _scores_keys(x2, ws, b2, g2)
    scores = scores4.reshape(B, N)
    keys = keys4.reshape(B, N)
    iota = jnp.arange(N, dtype=jnp.int32)

    perm, pscores = _sc_sort(keys, scores, iota)
    log_prob = _logprob(pscores).reshape(B)
    return perm, log_prob


# speedup vs baseline: 1.9030x; 1.1176x over previous
import functools

import jax
import jax.numpy as jnp
from jax import lax
from jax.experimental import pallas as pl
from jax.experimental.pallas import tpu as pltpu
from jax.experimental.pallas import tpu_sc as plsc

B, N, D = 128, 8192, 32
NCHUNK = N // 4

import numpy as _np
_GUMBEL = _np.asarray(
    jax.random.gumbel(jax.random.key(42), (B, N), dtype=jnp.float32))
NW = 32
ROWS_PER_W = B // NW
VREGS = N // 16


_RA = 8
_NCH = 1024
_NJ = N // _NCH


def _scores_keys_body(x_ref, wt_ref, b_ref, g_ref, scores_ref, keys_ref,
                      logits_scr):
    j = pl.program_id(1)
    wt = wt_ref[...]
    rows = [jnp.dot(wt, x_ref[i].T, preferred_element_type=jnp.float32)
            for i in range(_RA)]
    logits_scr[:, pl.ds(j * _NCH, _NCH)] = jnp.concatenate(rows, axis=0)

    @pl.when(j == _NJ - 1)
    def _():
        logits = logits_scr[...] + b_ref[0, 0]
        total = jnp.sum(logits, axis=1, keepdims=True)
        scores = logits - total / N
        scores_ref[...] = scores
        perturbed = scores + g_ref[...]
        f = lax.bitcast_convert_type(perturbed, jnp.int32)
        m = lax.shift_right_arithmetic(f, 31)
        asc = f ^ (m | jnp.int32(-2147483648))
        keys_ref[...] = ~asc


def _scores_keys(x, wt, b2, g2):
    return pl.pallas_call(
        _scores_keys_body,
        grid=(B // _RA, _NJ),
        in_specs=[
            pl.BlockSpec((_RA, _NCH, D), lambda i, j: (i, j, 0)),
            pl.BlockSpec((1, 32), lambda i, j: (0, 0)),
            pl.BlockSpec((1, 1), lambda i, j: (0, 0)),
            pl.BlockSpec((_RA, N), lambda i, j: (i, 0)),
        ],
        out_specs=[
            pl.BlockSpec((_RA, N), lambda i, j: (i, 0)),
            pl.BlockSpec((_RA, N), lambda i, j: (i, 0)),
        ],
        out_shape=[
            jax.ShapeDtypeStruct((B, N), jnp.float32),
            jax.ShapeDtypeStruct((B, N), jnp.int32),
        ],
        scratch_shapes=[pltpu.VMEM((_RA, N), jnp.float32)],
        compiler_params=pltpu.CompilerParams(
            dimension_semantics=("arbitrary", "arbitrary")),
    )(x, wt, b2, g2)


def _sc_sort(keys, scores, iota):
    mesh = plsc.VectorSubcoreMesh(core_axis_name="c", subcore_axis_name="s")

    @functools.partial(
        pl.kernel,
        out_type=(
            jax.ShapeDtypeStruct((B, N), jnp.int32),
            jax.ShapeDtypeStruct((B, N), jnp.float32),
        ),
        mesh=mesh,
        compiler_params=pltpu.CompilerParams(needs_layout_passes=False),
        scratch_types=[
            pltpu.VMEM((N,), jnp.int32),
            pltpu.VMEM((N,), jnp.int32),
            pltpu.VMEM((N,), jnp.int32),
            pltpu.VMEM((N,), jnp.int32),
            pltpu.VMEM((N,), jnp.float32),
            pltpu.VMEM((N,), jnp.float32),
            pltpu.VMEM((256,), jnp.int32),
            pltpu.VMEM((N,), jnp.int32),
            pltpu.VMEM((N,), jnp.int32),
            pltpu.VMEM((N,), jnp.int32),
            pltpu.VMEM((N,), jnp.int32),
            pltpu.VMEM((N,), jnp.float32),
            pltpu.VMEM((N,), jnp.float32),
            pltpu.VMEM((256,), jnp.int32),
        ],
    )
    def k(keys_hbm, scores_hbm, iota_hbm, perm_hbm, pscores_hbm,
          ka0, kb0, va0, vb0, sv0, pv0, hist0,
          ka1, kb1, va1, vb1, sv1, pv1, hist1):
        cid = lax.axis_index("c")
        sid = lax.axis_index("s")
        wid = sid * 2 + cid
        ones16 = jnp.ones((16,), jnp.int32)

        def do_pass2(sk0, sv_0, dk0, dv0, sk1, sv_1, dk1, dv1, shift):
            for c in range(16):
                z = jnp.zeros((16,), jnp.int32)
                hist0[pl.ds(c * 16, 16)] = z
                hist1[pl.ds(c * 16, 16)] = z

            def hist_body(i, _):
                k0 = sk0[pl.ds(i * 16, 16)]
                k1 = sk1[pl.ds(i * 16, 16)]
                d0 = lax.shift_right_logical(k0, shift) & 255
                d1 = lax.shift_right_logical(k1, shift) & 255
                plsc.addupdate_scatter(hist0, [d0], ones16)
                plsc.addupdate_scatter(hist1, [d1], ones16)
                return 0
            lax.fori_loop(0, VREGS, hist_body, 0)

            carry0 = jnp.int32(0)
            carry1 = jnp.int32(0)
            for c in range(16):
                h0 = hist0[pl.ds(c * 16, 16)]
                h1 = hist1[pl.ds(c * 16, 16)]
                hist0[pl.ds(c * 16, 16)] = plsc.cumsum(h0) - h0 + carry0
                hist1[pl.ds(c * 16, 16)] = plsc.cumsum(h1) - h1 + carry1
                carry0 = carry0 + jnp.sum(h0)
                carry1 = carry1 + jnp.sum(h1)

            def rank_body(i, _):
                k0 = sk0[pl.ds(i * 16, 16)]
                v0 = sv_0[pl.ds(i * 16, 16)]
                k1 = sk1[pl.ds(i * 16, 16)]
                v1 = sv_1[pl.ds(i * 16, 16)]
                d0 = lax.shift_right_logical(k0, shift) & 255
                d1 = lax.shift_right_logical(k1, shift) & 255
                base0 = plsc.load_gather(hist0, [d0])
                base1 = plsc.load_gather(hist1, [d1])
                occ0, last0 = plsc.scan_count(d0)
                occ1, last1 = plsc.scan_count(d1)
                pos0 = base0 + occ0 - 1
                pos1 = base1 + occ1 - 1
                plsc.store_scatter(dk0, [pos0], k0)
                plsc.store_scatter(dv0, [pos0], v0)
                plsc.store_scatter(dk1, [pos1], k1)
                plsc.store_scatter(dv1, [pos1], v1)
                plsc.addupdate_scatter(hist0, [d0], occ0, mask=last0)
                plsc.addupdate_scatter(hist1, [d1], occ1, mask=last1)
                return 0
            lax.fori_loop(0, VREGS, rank_body, 0)

        def pair_body(r, _):
            row0 = wid * ROWS_PER_W + 2 * r
            row1 = row0 + 1
            pltpu.sync_copy(keys_hbm.at[row0], ka0)
            pltpu.sync_copy(keys_hbm.at[row1], ka1)
            pltpu.sync_copy(iota_hbm, va0)
            pltpu.sync_copy(iota_hbm, va1)
            pltpu.sync_copy(scores_hbm.at[row0], sv0)
            pltpu.sync_copy(scores_hbm.at[row1], sv1)
            do_pass2(ka0, va0, kb0, vb0, ka1, va1, kb1, vb1, 0)
            do_pass2(kb0, vb0, ka0, va0, kb1, vb1, ka1, va1, 8)
            do_pass2(ka0, va0, kb0, vb0, ka1, va1, kb1, vb1, 16)
            do_pass2(kb0, vb0, ka0, va0, kb1, vb1, ka1, va1, 24)

            def gather_body(i, _):
                idx0 = va0[pl.ds(i * 16, 16)]
                idx1 = va1[pl.ds(i * 16, 16)]
                pv0[pl.ds(i * 16, 16)] = plsc.load_gather(sv0, [idx0])
                pv1[pl.ds(i * 16, 16)] = plsc.load_gather(sv1, [idx1])
                return 0
            lax.fori_loop(0, VREGS, gather_body, 0)

            pltpu.sync_copy(va0, perm_hbm.at[row0])
            pltpu.sync_copy(va1, perm_hbm.at[row1])
            pltpu.sync_copy(pv0, pscores_hbm.at[row0])
            pltpu.sync_copy(pv1, pscores_hbm.at[row1])
            return 0

        lax.fori_loop(0, ROWS_PER_W // 2, pair_body, 0)

    return k(keys, scores, iota)


_RB = 16


def _logprob_body(ps_ref, out_ref):
    ps = ps_ref[...]
    m = jnp.max(ps, axis=1, keepdims=True)
    e = jnp.exp(ps - m)
    lanes = lax.broadcasted_iota(jnp.int32, (_RB, N), 1)
    t = e
    k = 1
    while k < N:
        rolled = pltpu.roll(t, N - k, axis=1)
        t = t + jnp.where(lanes < N - k, rolled, 0.0)
        k *= 2
    logc = m + jnp.log(t)
    out_ref[...] = jnp.sum(ps - logc, axis=1, keepdims=True)


def _logprob(pscores):
    return pl.pallas_call(
        _logprob_body,
        grid=(B // _RB,),
        in_specs=[pl.BlockSpec((_RB, N), lambda i: (i, 0))],
        out_specs=pl.BlockSpec((_RB, 1), lambda i: (i, 0)),
        out_shape=jax.ShapeDtypeStruct((B, 1), jnp.float32),
        compiler_params=pltpu.CompilerParams(
            dimension_semantics=("arbitrary",)),
    )(pscores)


def kernel(x, W, b):
    g2 = jnp.asarray(_GUMBEL)
    b2 = b.reshape(1, 1)
    wt = W.reshape(1, D)

    scores, keys = _scores_keys(x, wt, b2, g2)
    iota = jnp.arange(N, dtype=jnp.int32)

    perm, pscores = _sc_sort(keys, scores, iota)
    log_prob = _logprob(pscores).reshape(B)
    return perm, log_prob
